# Initial kernel scaffold; baseline (speedup 1.0000x reference)
import jax
import jax.numpy as jnp
from jax.experimental import pallas as pl


def kernel(topo, struc, table, W1, b1, W2, b2, W3, b3):
    raise NotImplementedError("write your pallas kernel here")


# speedup vs baseline: 2.4050x; 2.4050x over previous
import functools

import jax
import jax.numpy as jnp
from jax import lax
from jax.experimental import pallas as pl
from jax.experimental.pallas import tpu as pltpu
from jax.experimental.pallas import tpu_sc as plsc

B = 16384
L = 50
EMBED = 64
NI = 128
DIM = 256

NW = 32
ROWS_PER_W = B // NW
CH = 16
NCHUNK = ROWS_PER_W // CH
IDX_PER_CH = CH * L
G_SIZE = 128
G_FULL = IDX_PER_CH // G_SIZE
G_TAIL = IDX_PER_CH % G_SIZE


def _sc_embedding_sum(table, topo_flat):
    mesh = plsc.VectorSubcoreMesh(core_axis_name="c", subcore_axis_name="s")

    @functools.partial(
        pl.kernel,
        out_type=jax.ShapeDtypeStruct((B, EMBED), jnp.float32),
        mesh=mesh,
        compiler_params=pltpu.CompilerParams(use_tc_tiling_on_sc=False),
        scratch_types=[
            pltpu.VMEM((IDX_PER_CH,), jnp.int32),
            pltpu.VMEM((IDX_PER_CH, EMBED), jnp.float32),
            pltpu.VMEM((CH, EMBED), jnp.float32),
            pltpu.SemaphoreType.DMA,
        ],
    )
    def sc_sum(table_hbm, topo_hbm, out_hbm, idx_v, rows_v, acc_v, sem):
        wid = lax.axis_index("s") * 2 + lax.axis_index("c")
        row0 = wid * ROWS_PER_W

        def chunk_body(c, carry):
            base = row0 + c * CH
            pltpu.sync_copy(topo_hbm.at[pl.ds(base * L, IDX_PER_CH)], idx_v)
            copies = []
            for g in range(G_FULL):
                copies.append(pltpu.async_copy(
                    table_hbm.at[idx_v.at[pl.ds(g * G_SIZE, G_SIZE)]],
                    rows_v.at[pl.ds(g * G_SIZE, G_SIZE)],
                    sem))
            if G_TAIL:
                copies.append(pltpu.async_copy(
                    table_hbm.at[idx_v.at[pl.ds(G_FULL * G_SIZE, G_TAIL)]],
                    rows_v.at[pl.ds(G_FULL * G_SIZE, G_TAIL)],
                    sem))
            for cp in copies:
                cp.wait()

            def row_body(r, rcarry):
                e0 = r * L
                accs = [jnp.zeros((16,), jnp.float32) for _ in range(4)]
                for l in range(L):
                    for j in range(4):
                        accs[j] = accs[j] + rows_v[e0 + l, pl.ds(j * 16, 16)]
                for j in range(4):
                    acc_v[r, pl.ds(j * 16, 16)] = accs[j]
                return rcarry

            lax.fori_loop(0, CH, row_body, 0)
            pltpu.sync_copy(acc_v, out_hbm.at[pl.ds(base, CH)])
            return carry

        lax.fori_loop(0, NCHUNK, chunk_body, 0)

    return sc_sum(table, topo_flat)


BB = 2048


def _mlp_body(topo_ref, sums_ref, struc_ref, w1a_ref, w1b_ref, b1_ref,
              w2_ref, b2_ref, w3_ref, b3_ref, out_ref):
    mask = (topo_ref[...] != 0).astype(jnp.float32)
    cnt = jnp.maximum(jnp.sum(mask, axis=1, keepdims=True), 1.0)
    emb = sums_ref[...] / cnt
    h = jnp.dot(emb, w1a_ref[...], preferred_element_type=jnp.float32)
    h = h + jnp.dot(struc_ref[...], w1b_ref[...],
                    preferred_element_type=jnp.float32)
    h = jnp.maximum(h + b1_ref[...], 0.0)
    h2 = jnp.dot(h, w2_ref[...], preferred_element_type=jnp.float32)
    h2 = jnp.maximum(h2 + b2_ref[...], 0.0)
    out = jnp.dot(h2, w3_ref[...], preferred_element_type=jnp.float32)
    out_ref[...] = out + b3_ref[...]


def _tc_mlp(topo, sums, struc, W1, b1, W2, b2, W3, b3):
    w1a = W1[:EMBED]
    w1b = W1[EMBED:]
    return pl.pallas_call(
        _mlp_body,
        grid=(B // BB,),
        in_specs=[
            pl.BlockSpec((BB, L), lambda i: (i, 0)),
            pl.BlockSpec((BB, EMBED), lambda i: (i, 0)),
            pl.BlockSpec((BB, NI), lambda i: (i, 0)),
            pl.BlockSpec((EMBED, 2 * DIM), lambda i: (0, 0)),
            pl.BlockSpec((NI, 2 * DIM), lambda i: (0, 0)),
            pl.BlockSpec((1, 2 * DIM), lambda i: (0, 0)),
            pl.BlockSpec((2 * DIM, DIM), lambda i: (0, 0)),
            pl.BlockSpec((1, DIM), lambda i: (0, 0)),
            pl.BlockSpec((DIM, 1), lambda i: (0, 0)),
            pl.BlockSpec((1, 1), lambda i: (0, 0)),
        ],
        out_specs=pl.BlockSpec((BB, 1), lambda i: (i, 0)),
        out_shape=jax.ShapeDtypeStruct((B, 1), jnp.float32),
    )(topo, sums, struc, w1a, w1b, b1.reshape(1, -1), W2,
      b2.reshape(1, -1), W3, b3.reshape(1, -1))


def kernel(topo, struc, table, W1, b1, W2, b2, W3, b3):
    sums = _sc_embedding_sum(table, topo.reshape(-1))
    return _tc_mlp(topo, sums, struc, W1, b1, W2, b2, W3, b3)
